# Initial kernel scaffold; baseline (speedup 1.0000x reference)
import jax
import jax.numpy as jnp
from jax.experimental import pallas as pl


def kernel(x, edge_index, Wp1, bp1, Wp2, bp2, Wp3, bp3, Wv1, bv1, Wv2, bv2, Wv3, bv3):
    raise NotImplementedError("write your pallas kernel here")


# speedup vs baseline: 11.9549x; 11.9549x over previous
import functools

import jax
import jax.numpy as jnp
from jax import lax
from jax.experimental import pallas as pl
from jax.experimental.pallas import tpu as pltpu
from jax.experimental.pallas import tpu_sc as plsc

NC = 2
NS = 16
NW = NC * NS
CHUNK = 80

_mesh = lambda: plsc.VectorSubcoreMesh(core_axis_name="c", subcore_axis_name="s",
                                       num_cores=NC, num_subcores=NS)


def _zero_fill(zbuf, rows, width):
    def st(i, _):
        r = i // (width // 16)
        k = i % (width // 16)
        zbuf[r, pl.ds(k * 16, 16)] = jnp.zeros((16,), jnp.float32)
        return 0
    lax.fori_loop(0, rows * (width // 16), st, 0)


def _sc_apply(table, src, dst, n, width):
    e = src.shape[0]
    ew = e // NW
    steps = ew // CHUNK
    npad = ((n + NS * 8 - 1) // (NS * 8)) * (NS * 8)
    rows_t = npad // NS
    zrows = 8
    assert ew % CHUNK == 0 and rows_t % zrows == 0

    @functools.partial(
        pl.kernel,
        out_type=jax.ShapeDtypeStruct((NC, npad, width), jnp.float32),
        mesh=_mesh(),
        scratch_types=[
            pltpu.VMEM((CHUNK,), jnp.int32),
            pltpu.VMEM((CHUNK,), jnp.int32),
            pltpu.VMEM((CHUNK, width), jnp.float32),
            pltpu.VMEM((zrows, width), jnp.float32),
            pltpu.VMEM_SHARED((npad, width), jnp.float32),
            pltpu.SemaphoreType.DMA,
        ],
        compiler_params=pltpu.CompilerParams(
            use_tc_tiling_on_sc=(width % 128 == 0)),
    )
    def k(src_hbm, dst_hbm, table_hbm, out_hbm, src_v, dst_v, rows_v, zbuf,
          acc, sem):
        c = lax.axis_index("c")
        s = lax.axis_index("s")
        wid = s * NC + c

        _zero_fill(zbuf, zrows, width)

        def zc(i, _):
            pltpu.sync_copy(zbuf, acc.at[pl.ds(s * rows_t + i * zrows, zrows)])
            return 0
        lax.fori_loop(0, rows_t // zrows, zc, 0)
        plsc.subcore_barrier()

        def body(i, _):
            base = pl.multiple_of(wid * ew + i * CHUNK, 8)
            pltpu.sync_copy(src_hbm.at[pl.ds(base, CHUNK)], src_v)
            pltpu.sync_copy(dst_hbm.at[pl.ds(base, CHUNK)], dst_v)
            pltpu.async_copy(table_hbm.at[src_v], rows_v, sem).wait()
            pltpu.sync_copy(rows_v, acc.at[dst_v], add=True)
            return 0
        lax.fori_loop(0, steps, body, 0)
        plsc.subcore_barrier()

        pltpu.sync_copy(acc.at[pl.ds(s * rows_t, rows_t)],
                        out_hbm.at[c, pl.ds(s * rows_t, rows_t)])

    return k(src, dst, table)[:, :n]


def _sc_degree(dst, n):
    e = dst.shape[0]
    ew = e // NW
    steps = ew // CHUNK
    npad = ((n + NS * 8 - 1) // (NS * 8)) * (NS * 8)
    rows_t = npad // NS
    zrows = 8
    width = 16

    @functools.partial(
        pl.kernel,
        out_type=jax.ShapeDtypeStruct((NC, npad, width), jnp.float32),
        mesh=_mesh(),
        scratch_types=[
            pltpu.VMEM((CHUNK,), jnp.int32),
            pltpu.VMEM((CHUNK, width), jnp.float32),
            pltpu.VMEM((zrows, width), jnp.float32),
            pltpu.VMEM_SHARED((npad, width), jnp.float32),
        ],
        compiler_params=pltpu.CompilerParams(use_tc_tiling_on_sc=False),
    )
    def k(dst_hbm, out_hbm, dst_v, ones_v, zbuf, acc):
        c = lax.axis_index("c")
        s = lax.axis_index("s")
        wid = s * NC + c

        _zero_fill(zbuf, zrows, width)

        def of(i, _):
            ones_v[i, pl.ds(0, 16)] = jnp.ones((16,), jnp.float32)
            return 0
        lax.fori_loop(0, CHUNK, of, 0)

        def zc(i, _):
            pltpu.sync_copy(zbuf, acc.at[pl.ds(s * rows_t + i * zrows, zrows)])
            return 0
        lax.fori_loop(0, rows_t // zrows, zc, 0)
        plsc.subcore_barrier()

        def body(i, _):
            base = pl.multiple_of(wid * ew + i * CHUNK, 8)
            pltpu.sync_copy(dst_hbm.at[pl.ds(base, CHUNK)], dst_v)
            pltpu.sync_copy(ones_v, acc.at[dst_v], add=True)
            return 0
        lax.fori_loop(0, steps, body, 0)
        plsc.subcore_barrier()

        pltpu.sync_copy(acc.at[pl.ds(s * rows_t, rows_t)],
                        out_hbm.at[c, pl.ds(s * rows_t, rows_t)])

    return k(dst)[:, :n]


_RB = 2000


def _row_spec(width):
    return pl.BlockSpec((_RB, width), lambda i: (i, 0))


def _part_spec(width):
    return pl.BlockSpec((NC, _RB, width), lambda i: (0, i, 0))


def _full_spec(shape):
    return pl.BlockSpec(shape, lambda i: tuple(0 for _ in shape))


def _tc_prep(degp, x):
    n, d = x.shape

    def body(degp_ref, x_ref, dinv_ref, xs_ref):
        deg = degp_ref[0, :, 0:1] + degp_ref[1, :, 0:1] + 1.0
        dinv = lax.rsqrt(deg)
        dinv_ref[...] = dinv
        xs_ref[...] = x_ref[...] * dinv

    return pl.pallas_call(
        body,
        grid=(n // _RB,),
        in_specs=[_part_spec(16), _row_spec(d)],
        out_specs=[_row_spec(1), _row_spec(d)],
        out_shape=[jax.ShapeDtypeStruct((n, 1), jnp.float32),
                   jax.ShapeDtypeStruct((n, d), jnp.float32)],
    )(degp, x)


def _tc_layer1(p, xs0, dinv, Wp1, bp1, Wv1, bv1):
    n, d = xs0.shape
    h = Wp1.shape[1]

    def body(p_ref, xs_ref, dinv_ref, wp_ref, bp_ref, wv_ref, bv_ref,
             xa_ref, xv_ref):
        dv = dinv_ref[...]
        z = dv * (p_ref[0] + p_ref[1] + xs_ref[...])
        a1 = jnp.maximum(jnp.dot(z, wp_ref[...],
                                 preferred_element_type=jnp.float32)
                         + bp_ref[...], 0.0)
        v1 = jnp.maximum(jnp.dot(z, wv_ref[...],
                                 preferred_element_type=jnp.float32)
                         + bv_ref[...], 0.0)
        xa_ref[...] = dv * a1
        xv_ref[...] = dv * v1

    return pl.pallas_call(
        body,
        grid=(n // _RB,),
        in_specs=[_part_spec(d), _row_spec(d), _row_spec(1),
                  _full_spec((d, h)), _full_spec((1, h)),
                  _full_spec((d, h)), _full_spec((1, h))],
        out_specs=[_row_spec(h), _row_spec(h)],
        out_shape=[jax.ShapeDtypeStruct((n, h), jnp.float32),
                   jax.ShapeDtypeStruct((n, h), jnp.float32)],
    )(p, xs0, dinv, Wp1, bp1.reshape(1, -1), Wv1, bv1.reshape(1, -1))


def _tc_layer23(pa, pv, xa1, xv1, dinv, Wp2, bp2, Wp3, Wv2, bv2, Wv3):
    n, h = xa1.shape
    out_p = Wp3.shape[1]

    def body(pa_ref, pv_ref, xa_ref, xv_ref, dinv_ref,
             wp2_ref, bp2_ref, wp3_ref, wv2_ref, bv2_ref, wv3_ref, hcat_ref):
        dv = dinv_ref[...]
        za = dv * (pa_ref[0] + pa_ref[1] + xa_ref[...])
        a2 = jnp.maximum(jnp.dot(za, wp2_ref[...],
                                 preferred_element_type=jnp.float32)
                         + bp2_ref[...], 0.0)
        hp = jnp.dot(a2, wp3_ref[...], preferred_element_type=jnp.float32)
        zv = dv * (pv_ref[0] + pv_ref[1] + xv_ref[...])
        v2 = jnp.maximum(jnp.dot(zv, wv2_ref[...],
                                 preferred_element_type=jnp.float32)
                         + bv2_ref[...], 0.0)
        hv = jnp.dot(v2, wv3_ref[...], preferred_element_type=jnp.float32)
        pad = jnp.zeros((hp.shape[0], 32 - out_p - 1), jnp.float32)
        hcat_ref[...] = dv * jnp.concatenate([hp, hv, pad], axis=1)

    return pl.pallas_call(
        body,
        grid=(n // _RB,),
        in_specs=[_part_spec(h), _part_spec(h), _row_spec(h), _row_spec(h),
                  _row_spec(1),
                  _full_spec((h, h)), _full_spec((1, h)),
                  _full_spec((h, out_p)),
                  _full_spec((h, h)), _full_spec((1, h)),
                  _full_spec((h, 1))],
        out_specs=[_row_spec(32)],
        out_shape=[jax.ShapeDtypeStruct((n, 32), jnp.float32)],
    )(pa, pv, xa1, xv1, dinv, Wp2, bp2.reshape(1, -1), Wp3,
      Wv2, bv2.reshape(1, -1), Wv3)[0]


def _tc_final(pc, hcat, dinv, bp3, bv3, out_p):
    n = hcat.shape[0]

    def body(pc_ref, hcat_ref, dinv_ref, bp3_ref, bv3_ref, lg_ref, vl_ref):
        cfull = dinv_ref[...] * (pc_ref[0] + pc_ref[1] + hcat_ref[...])
        lg_ref[...] = cfull[:, :out_p] + bp3_ref[...]
        vl_ref[...] = cfull[:, out_p:out_p + 1] + bv3_ref[...]

    return pl.pallas_call(
        body,
        grid=(n // _RB,),
        in_specs=[_part_spec(32), _row_spec(32), _row_spec(1),
                  _full_spec((1, out_p)), _full_spec((1, 1))],
        out_specs=[_row_spec(out_p), _row_spec(1)],
        out_shape=[jax.ShapeDtypeStruct((n, out_p), jnp.float32),
                   jax.ShapeDtypeStruct((n, 1), jnp.float32)],
    )(pc, hcat, dinv, bp3.reshape(1, -1), bv3.reshape(1, -1))


def kernel(x, edge_index, Wp1, bp1, Wp2, bp2, Wp3, bp3, Wv1, bv1, Wv2, bv2,
           Wv3, bv3):
    n, d = x.shape
    out_p = Wp3.shape[1]
    src = edge_index[0]
    dst = edge_index[1]

    degp = _sc_degree(dst, n)
    dinv, xs0 = _tc_prep(degp, x)

    p0 = _sc_apply(xs0, src, dst, n, d)
    xa1, xv1 = _tc_layer1(p0, xs0, dinv, Wp1, bp1, Wv1, bv1)

    pa = _sc_apply(xa1, src, dst, n, d)
    pv = _sc_apply(xv1, src, dst, n, d)
    hcat = _tc_layer23(pa, pv, xa1, xv1, dinv, Wp2, bp2, Wp3, Wv2, bv2, Wv3)

    pc = _sc_apply(hcat, src, dst, n, 32)
    logits, value = _tc_final(pc, hcat, dinv, bp3, bv3, out_p)
    return (logits, value)
